# baseline (speedup 1.0000x reference)
import functools

import jax
import jax.numpy as jnp
from jax import lax
from jax.experimental import pallas as pl
from jax.experimental.pallas import tpu as pltpu
from jax.experimental.pallas import tpu_sc as plsc

BATCH = 16384
F = 26
D = 16
VOCAB = 100000

NC = 2
NS = 16
NW = NC * NS

CHUNK_B = 128
RPC = CHUNK_B * F
IDX_MINOR = 128
IDX_ROWS = RPC // IDX_MINOR
CHUNKS = BATCH // CHUNK_B
CHUNKS_PER_W = CHUNKS // NW


def _body(x_hbm, off_hbm, tab_hbm, out_hbm, xv, offv, rows0, rows1,
          gsem, wsem0, wsem1):
  cid = lax.axis_index("c")
  sid = lax.axis_index("s")
  wid = sid * NC + cid

  pltpu.sync_copy(off_hbm, offv)

  rows_bufs = (rows0, rows1)
  wsems = (wsem0, wsem1)

  for c in range(CHUNKS_PER_W):
    rows = rows_bufs[c & 1]
    wsem = wsems[c & 1]
    chunk = wid * CHUNKS_PER_W + c

    pltpu.sync_copy(x_hbm.at[chunk], xv)

    def add_body(r, carry):
      for j in range(IDX_MINOR // 16):
        sl = pl.ds(j * 16, 16)
        xv[r, sl] = xv[r, sl] + offv[r, sl]
      return carry

    lax.fori_loop(0, IDX_ROWS, add_body, 0)

    if c >= 2:
      prev = chunk - 2
      pltpu.make_async_copy(
          rows, out_hbm.at[pl.ds(prev * RPC, RPC)], wsem).wait()

    def fire(r, carry):
      pltpu.make_async_copy(
          tab_hbm.at[xv.at[r]], rows.at[pl.ds(r * IDX_MINOR, IDX_MINOR)],
          gsem).start()
      return carry

    lax.fori_loop(0, IDX_ROWS, fire, 0)

    pltpu.make_async_copy(tab_hbm.at[pl.ds(0, RPC)], rows, gsem).wait()

    pltpu.make_async_copy(
        rows, out_hbm.at[pl.ds(chunk * RPC, RPC)], wsem).start()

  for c in (CHUNKS_PER_W - 2, CHUNKS_PER_W - 1):
    chunk = wid * CHUNKS_PER_W + c
    pltpu.make_async_copy(
        rows_bufs[c & 1], out_hbm.at[pl.ds(chunk * RPC, RPC)],
        wsems[c & 1]).wait()


@jax.jit
def _gather_all(x_chunks, off, tab_flat):
  mesh = plsc.VectorSubcoreMesh(core_axis_name="c", subcore_axis_name="s")
  kern = functools.partial(
      pl.kernel,
      mesh=mesh,
      compiler_params=pltpu.CompilerParams(use_tc_tiling_on_sc=False),
      out_type=jax.ShapeDtypeStruct((BATCH * F, D), jnp.float32),
      scratch_types=[
          pltpu.VMEM((IDX_ROWS, IDX_MINOR), jnp.int32),
          pltpu.VMEM((IDX_ROWS, IDX_MINOR), jnp.int32),
          pltpu.VMEM((RPC, D), jnp.float32),
          pltpu.VMEM((RPC, D), jnp.float32),
          pltpu.SemaphoreType.DMA,
          pltpu.SemaphoreType.DMA,
          pltpu.SemaphoreType.DMA,
      ],
  )(_body)
  return kern(x_chunks, off, tab_flat)


def kernel(X_sparse, X_dense, tables):
  x_chunks = X_sparse.reshape(CHUNKS, IDX_ROWS, IDX_MINOR)
  off = jnp.tile(jnp.arange(F, dtype=jnp.int32) * VOCAB,
                 CHUNK_B).reshape(IDX_ROWS, IDX_MINOR)
  tab_flat = tables.reshape(F * VOCAB, D)
  out = _gather_all(x_chunks, off, tab_flat)
  return out.reshape(BATCH, F, D), X_dense


# speedup vs baseline: 7.6707x; 7.6707x over previous
import functools

import jax
import jax.numpy as jnp
from jax import lax
from jax.experimental import pallas as pl
from jax.experimental.pallas import tpu as pltpu
from jax.experimental.pallas import tpu_sc as plsc

BATCH = 16384
F = 26
D = 16
VOCAB = 100000

NC = 2
NS = 16
NW = NC * NS
TASKS = F * D
TASKS_PER_W = TASKS // NW
CB = 8192


def _body(xT_hbm, tab_hbm, out_hbm, xv, rowv, outv):
  cid = lax.axis_index("c")
  sid = lax.axis_index("s")
  wid = sid * NC + cid
  t0 = wid * TASKS_PER_W

  for k in range(TASKS_PER_W):
    t = t0 + k
    f = t // D
    d = t % D

    if k == 0:
      pltpu.sync_copy(xT_hbm.at[f], xv)
    else:
      prev_f = (t - 1) // D

      @pl.when(f != prev_f)
      def _reload():
        pltpu.sync_copy(xT_hbm.at[f], xv)

    pltpu.sync_copy(tab_hbm.at[f, d], rowv)

    for ch in range(BATCH // CB):

      def gloop(i, carry, ch=ch):
        idx = xv[pl.ds(ch * CB + i * 16, 16)]
        outv[pl.ds(i * 16, 16)] = plsc.load_gather(rowv, [idx])
        return carry

      lax.fori_loop(0, CB // 16, gloop, 0)
      pltpu.sync_copy(outv, out_hbm.at[f, d, pl.ds(ch * CB, CB)])


@jax.jit
def _gather_all(xT, tabT):
  mesh = plsc.VectorSubcoreMesh(core_axis_name="c", subcore_axis_name="s")
  kern = functools.partial(
      pl.kernel,
      mesh=mesh,
      compiler_params=pltpu.CompilerParams(
          use_tc_tiling_on_sc=True, needs_layout_passes=False),
      out_type=jax.ShapeDtypeStruct((F, D, BATCH), jnp.float32),
      scratch_types=[
          pltpu.VMEM((BATCH,), jnp.int32),
          pltpu.VMEM((VOCAB,), jnp.float32),
          pltpu.VMEM((CB,), jnp.float32),
      ],
  )(_body)
  return kern(xT, tabT)


def kernel(X_sparse, X_dense, tables):
  xT = X_sparse.T
  tabT = jnp.transpose(tables, (0, 2, 1))
  outT = _gather_all(xT, tabT)
  return jnp.transpose(outT, (2, 0, 1)), X_dense


# speedup vs baseline: 10.0453x; 1.3096x over previous
import functools

import jax
import jax.numpy as jnp
from jax import lax
from jax.experimental import pallas as pl
from jax.experimental.pallas import tpu as pltpu
from jax.experimental.pallas import tpu_sc as plsc

BATCH = 16384
F = 26
D = 16
VOCAB = 100000

NC = 2
NS = 16
NW = NC * NS
TASKS = F * D
TASKS_PER_W = TASKS // NW
CB = 4096
NCH = BATCH // CB
UNROLL = 8


def _body(xT_hbm, tab_hbm, out_hbm, xv, rowv, outv0, outv1, rsem, wsem0, wsem1):
  cid = lax.axis_index("c")
  sid = lax.axis_index("s")
  wid = sid * NC + cid
  t0 = wid * TASKS_PER_W

  outvs = (outv0, outv1)
  wsems = (wsem0, wsem1)

  def row_copy(k):
    t = t0 + k
    return pltpu.make_async_copy(tab_hbm.at[t // D, t % D], rowv, rsem)

  row_copy(0).start()

  nchunks = 0
  for k in range(TASKS_PER_W):
    t = t0 + k
    f = t // D
    d = t % D

    if k == 0:
      pltpu.sync_copy(xT_hbm.at[f], xv)
    else:
      prev_f = (t - 1) // D

      @pl.when(f != prev_f)
      def _reload():
        pltpu.sync_copy(xT_hbm.at[f], xv)

    row_copy(k).wait()

    for ch in range(NCH):
      j = ch & 1
      outv = outvs[j]
      if nchunks >= 2:
        pltpu.make_async_copy(
            out_hbm.at[f, d, pl.ds(0, CB)], outv, wsems[j]).wait()

      def gloop(i, carry, ch=ch, outv=outv):
        base = ch * CB + i * (UNROLL * 16)
        obase = i * (UNROLL * 16)
        for u in range(UNROLL):
          idx = xv[pl.ds(base + u * 16, 16)]
          outv[pl.ds(obase + u * 16, 16)] = plsc.load_gather(rowv, [idx])
        return carry

      lax.fori_loop(0, CB // (UNROLL * 16), gloop, 0)
      pltpu.make_async_copy(
          outv, out_hbm.at[f, d, pl.ds(ch * CB, CB)], wsems[j]).start()
      nchunks += 1

    if k + 1 < TASKS_PER_W:
      row_copy(k + 1).start()

  tl = t0 + TASKS_PER_W - 1
  for j in range(2):
    pltpu.make_async_copy(
        out_hbm.at[tl // D, tl % D, pl.ds(0, CB)], outvs[j], wsems[j]).wait()


@jax.jit
def _gather_all(xT, tabT):
  mesh = plsc.VectorSubcoreMesh(core_axis_name="c", subcore_axis_name="s")
  kern = functools.partial(
      pl.kernel,
      mesh=mesh,
      compiler_params=pltpu.CompilerParams(
          use_tc_tiling_on_sc=True, needs_layout_passes=False),
      out_type=jax.ShapeDtypeStruct((F, D, BATCH), jnp.float32),
      scratch_types=[
          pltpu.VMEM((BATCH,), jnp.int32),
          pltpu.VMEM((VOCAB,), jnp.float32),
          pltpu.VMEM((CB,), jnp.float32),
          pltpu.VMEM((CB,), jnp.float32),
          pltpu.SemaphoreType.DMA,
          pltpu.SemaphoreType.DMA,
          pltpu.SemaphoreType.DMA,
      ],
  )(_body)
  return kern(xT, tabT)


def kernel(X_sparse, X_dense, tables):
  xT = X_sparse.T
  tabT = jnp.transpose(tables, (0, 2, 1))
  outT = _gather_all(xT, tabT)
  return jnp.transpose(outT, (2, 0, 1)), X_dense


# speedup vs baseline: 13.6013x; 1.3540x over previous
import functools

import jax
import jax.numpy as jnp
from jax import lax
from jax.experimental import pallas as pl
from jax.experimental.pallas import tpu as pltpu
from jax.experimental.pallas import tpu_sc as plsc

BATCH = 16384
F = 26
D = 16
VOCAB = 100000

NC = 2
NS = 16
NW = NC * NS
TASKS = F * D
TASKS_PER_W = TASKS // NW
CB = 4096
NCH = BATCH // CB
UNROLL = 8


def _body(xT_hbm, tab_hbm, out_hbm, xv, rowv, outv0, outv1, rsem, wsem0, wsem1):
  cid = lax.axis_index("c")
  sid = lax.axis_index("s")
  wid = sid * NC + cid
  t0 = wid * TASKS_PER_W

  outvs = (outv0, outv1)
  wsems = (wsem0, wsem1)

  def row_copy(k):
    t = t0 + k
    return pltpu.make_async_copy(tab_hbm.at[t // D, t % D], rowv, rsem)

  row_copy(0).start()

  nchunks = 0
  for k in range(TASKS_PER_W):
    t = t0 + k
    f = t // D
    d = t % D

    if k == 0:
      pltpu.sync_copy(xT_hbm.at[f], xv)
    else:
      prev_f = (t - 1) // D

      @pl.when(f != prev_f)
      def _reload():
        pltpu.sync_copy(xT_hbm.at[f], xv)

    row_copy(k).wait()

    for ch in range(NCH):
      j = ch & 1
      outv = outvs[j]
      if nchunks >= 2:
        pltpu.make_async_copy(
            out_hbm.at[f, d, pl.ds(0, CB)], outv, wsems[j]).wait()

      def _gather_chunk(ch, outv):
        @plsc.parallel_loop(0, CB // 16, step=1, unroll=UNROLL)
        def _g(i):
          idx = xv[pl.ds(ch * CB + i * 16, 16)]
          outv[pl.ds(i * 16, 16)] = plsc.load_gather(rowv, [idx])

      _gather_chunk(ch, outv)
      pltpu.make_async_copy(
          outv, out_hbm.at[f, d, pl.ds(ch * CB, CB)], wsems[j]).start()
      nchunks += 1

    if k + 1 < TASKS_PER_W:
      row_copy(k + 1).start()

  tl = t0 + TASKS_PER_W - 1
  for j in range(2):
    pltpu.make_async_copy(
        out_hbm.at[tl // D, tl % D, pl.ds(0, CB)], outvs[j], wsems[j]).wait()


@jax.jit
def _gather_all(xT, tabT):
  mesh = plsc.VectorSubcoreMesh(core_axis_name="c", subcore_axis_name="s")
  kern = functools.partial(
      pl.kernel,
      mesh=mesh,
      compiler_params=pltpu.CompilerParams(
          use_tc_tiling_on_sc=True, needs_layout_passes=False),
      out_type=jax.ShapeDtypeStruct((F, D, BATCH), jnp.float32),
      scratch_types=[
          pltpu.VMEM((BATCH,), jnp.int32),
          pltpu.VMEM((VOCAB,), jnp.float32),
          pltpu.VMEM((CB,), jnp.float32),
          pltpu.VMEM((CB,), jnp.float32),
          pltpu.SemaphoreType.DMA,
          pltpu.SemaphoreType.DMA,
          pltpu.SemaphoreType.DMA,
      ],
  )(_body)
  return kern(xT, tabT)


def kernel(X_sparse, X_dense, tables):
  xT = X_sparse.T
  tabT = jnp.transpose(tables, (0, 2, 1))
  outT = _gather_all(xT, tabT)
  return jnp.transpose(outT, (2, 0, 1)), X_dense
